# baseline (speedup 1.0000x reference)
import functools

import jax
import jax.numpy as jnp
from jax.experimental import pallas as pl


def _lap_valid(a):
    R = a.shape[0]
    w2 = a[0:R - 1] + a[1:R]
    w4 = w2[0:R - 3] + w2[2:R - 1]
    w8 = w4[0:R - 7] + w4[4:R - 3]
    w9 = w8[0:R - 8] + a[8:R]
    return (a[4:R - 4] - w9) * 0.125


def _lap_circ(a):
    return _lap_valid(jnp.concatenate([a[-4:], a, a[:4]], axis=0))


def _dot(a, b):
    return jnp.dot(a, b, preferred_element_type=jnp.float32)


def _enc0_body(x_ref, w_ref, b_ref, o_ref, p_ref, *, cn, n):
    c = pl.program_id(0)
    nb = n // cn
    s = c * cn
    lo = jnp.where(c == 0, n - 8, s - 8)
    hi = jnp.where(c == nb - 1, 0, s + cn)
    win = jnp.concatenate(
        [x_ref[pl.ds(lo, 8), :], x_ref[pl.ds(s, cn), :], x_ref[pl.ds(hi, 8), :]],
        axis=0)
    x1w = _lap_valid(win)
    x0c = win[8:cn + 8]
    x2c = 2.0 * _lap_valid(x1w) - x0c
    acc = _dot(x0c, w_ref[0]) + _dot(x1w[4:cn + 4], w_ref[1]) + _dot(x2c, w_ref[2])
    acc = jnp.maximum(acc + b_ref[0, :][None, :], 0.0)
    o_ref[...] = acc
    f = acc.shape[1]
    p = acc.reshape(cn // 4, 4, f)
    p_ref[...] = (p[:, 0] + p[:, 1] + p[:, 2] + p[:, 3]) * 0.25


def _mid_body(p0_ref, w1_ref, b1_ref, w2_ref, b2_ref, w3_ref, b3_ref,
              w4_ref, b4_ref, d0u_ref, d0s_ref, db0_ref, d1u_ref, d1s_ref,
              db1_ref, d2u_ref, d2s_ref, db2_ref, d2_ref):

    def conv_before(x, w_ref, b_ref):
        x1 = _lap_circ(x)
        x2 = 2.0 * _lap_circ(x1) - x
        acc = _dot(x, w_ref[0]) + _dot(x1, w_ref[1]) + _dot(x2, w_ref[2])
        return jnp.maximum(acc + b_ref[0, :][None, :], 0.0)

    def pool(x):
        m, f = x.shape
        p = x.reshape(m // 4, 4, f)
        return (p[:, 0] + p[:, 1] + p[:, 2] + p[:, 3]) * 0.25

    def conv_after_unpool(hc, skip, wu_ref, ws_ref, b_ref):
        ys = []
        for k in range(3):
            u = jnp.repeat(_dot(hc, wu_ref[k]), 4, axis=0)
            ys.append(u + _dot(skip, ws_ref[k]))
        t = _lap_circ(ys[2])
        h = ys[0] - ys[2] + _lap_circ(ys[1] + 2.0 * t)
        return jnp.maximum(h + b_ref[0, :][None, :], 0.0)

    h = conv_before(p0_ref[...], w1_ref, b1_ref)
    skip1 = h
    h = conv_before(pool(h), w2_ref, b2_ref)
    skip2 = h
    h = conv_before(pool(h), w3_ref, b3_ref)
    skip3 = h
    h = conv_before(pool(h), w4_ref, b4_ref)
    h = conv_after_unpool(h, skip3, d0u_ref, d0s_ref, db0_ref)
    h = conv_after_unpool(h, skip2, d1u_ref, d1s_ref, db1_ref)
    h = conv_after_unpool(h, skip1, d2u_ref, d2s_ref, db2_ref)
    d2_ref[...] = h


def _fine_body(d2_ref, s0_ref, wu_ref, ws_ref, b3_ref, wo_ref, bo_ref,
               out_ref, *, cn, n):
    c = pl.program_id(0)
    nb = n // cn
    s = c * cn
    nc = n // 4
    cnc = cn // 4
    sc = c * cnc
    lo = jnp.where(c == 0, n - 16, s - 16)
    hi = jnp.where(c == nb - 1, 0, s + cn)
    sw = jnp.concatenate(
        [s0_ref[pl.ds(lo, 16), :], s0_ref[pl.ds(s, cn), :],
         s0_ref[pl.ds(hi, 16), :]], axis=0)
    loc = jnp.where(c == 0, nc - 4, sc - 4)
    hic = jnp.where(c == nb - 1, 0, sc + cnc)
    dw = jnp.concatenate(
        [d2_ref[pl.ds(loc, 4), :], d2_ref[pl.ds(sc, cnc), :],
         d2_ref[pl.ds(hic, 4), :]], axis=0)
    ys = []
    for k in range(3):
        u = jnp.repeat(_dot(dw, wu_ref[k]), 4, axis=0)
        ys.append(u + _dot(sw, ws_ref[k]))
    t = _lap_valid(ys[2])
    u1 = ys[1][4:cn + 28] + 2.0 * t
    h = ys[0][8:cn + 24] - ys[2][8:cn + 24] + _lap_valid(u1)
    h = jnp.maximum(h + b3_ref[0, :][None, :], 0.0)
    z0 = _dot(h[8:cn + 8], wo_ref[0])
    z1 = _dot(h[4:cn + 12], wo_ref[1])
    z2 = _dot(h, wo_ref[2])
    t2 = _lap_valid(z2)
    res = z0 - z2[8:cn + 8] + _lap_valid(z1 + 2.0 * t2)
    out_ref[...] = res + bo_ref[0, :][None, :]


def _pack(x):
    b, n, f = x.shape
    return jnp.transpose(x, (1, 0, 2)).reshape(n, b * f)


def _unpack(xp, f):
    n = xp.shape[0]
    return jnp.transpose(xp.reshape(n, 2, f), (1, 0, 2))


def _pack_w(w):
    eye = jnp.eye(2, dtype=w.dtype)
    return jnp.stack([jnp.kron(eye, w[k]) for k in range(w.shape[0])])


def _pack_b(b):
    return jnp.concatenate([b, b]).reshape(1, -1)


def kernel(x, enc_w0, enc_b0, enc_w1, enc_b1, enc_w2, enc_b2, enc_w3, enc_b3,
           enc_w4, enc_b4, dec_w0, dec_b0, dec_w1, dec_b1, dec_w2, dec_b2,
           dec_w3, dec_b3, out_w, out_b):
    n = 12288
    cn = 2048
    xp = _pack(x)

    skip0, p0 = pl.pallas_call(
        functools.partial(_enc0_body, cn=cn, n=n),
        grid=(n // cn,),
        in_specs=[
            pl.BlockSpec((n, 4), lambda c: (0, 0)),
            pl.BlockSpec((3, 4, 64), lambda c: (0, 0, 0)),
            pl.BlockSpec((1, 64), lambda c: (0, 0)),
        ],
        out_shape=[jax.ShapeDtypeStruct((n, 64), jnp.float32),
                   jax.ShapeDtypeStruct((n // 4, 64), jnp.float32)],
        out_specs=[pl.BlockSpec((cn, 64), lambda c: (c, 0)),
                   pl.BlockSpec((cn // 4, 64), lambda c: (c, 0))],
    )(xp, _pack_w(enc_w0), _pack_b(enc_b0))

    mid_args = (
        p0,
        _pack_w(enc_w1), _pack_b(enc_b1),
        _pack_w(enc_w2), _pack_b(enc_b2),
        _pack_w(enc_w3), _pack_b(enc_b3),
        _pack_w(enc_w4), _pack_b(enc_b4),
        _pack_w(dec_w0[:, :512]), _pack_w(dec_w0[:, 512:]), _pack_b(dec_b0),
        _pack_w(dec_w1[:, :256]), _pack_w(dec_w1[:, 256:]), _pack_b(dec_b1),
        _pack_w(dec_w2[:, :128]), _pack_w(dec_w2[:, 128:]), _pack_b(dec_b2),
    )
    d2 = pl.pallas_call(
        _mid_body,
        out_shape=jax.ShapeDtypeStruct((3072, 128), jnp.float32),
    )(*mid_args)

    outp = pl.pallas_call(
        functools.partial(_fine_body, cn=cn, n=n),
        grid=(n // cn,),
        in_specs=[
            pl.BlockSpec((n // 4, 128), lambda c: (0, 0)),
            pl.BlockSpec((n, 64), lambda c: (0, 0)),
            pl.BlockSpec((3, 128, 64), lambda c: (0, 0, 0)),
            pl.BlockSpec((3, 64, 64), lambda c: (0, 0, 0)),
            pl.BlockSpec((1, 64), lambda c: (0, 0)),
            pl.BlockSpec((3, 64, 4), lambda c: (0, 0, 0)),
            pl.BlockSpec((1, 4), lambda c: (0, 0)),
        ],
        out_shape=jax.ShapeDtypeStruct((n, 4), jnp.float32),
        out_specs=pl.BlockSpec((cn, 4), lambda c: (c, 0)),
    )(d2, skip0, _pack_w(dec_w3[:, :64]), _pack_w(dec_w3[:, 64:]),
      _pack_b(dec_b3), _pack_w(out_w), _pack_b(out_b))

    return _unpack(outp, 2)


# speedup vs baseline: 1.6027x; 1.6027x over previous
import functools

import jax
import jax.numpy as jnp
from jax.experimental import pallas as pl


def _lap_valid(a):
    R = a.shape[0]
    w2 = a[0:R - 1] + a[1:R]
    w4 = w2[0:R - 3] + w2[2:R - 1]
    w8 = w4[0:R - 7] + w4[4:R - 3]
    w9 = w8[0:R - 8] + a[8:R]
    return (a[4:R - 4] - w9) * 0.125


def _lap_circ(a):
    return _lap_valid(jnp.concatenate([a[-4:], a, a[:4]], axis=0))


def _dot(a, b):
    return jnp.dot(a, b, preferred_element_type=jnp.float32)


def _enc0_body(x_ref, w_ref, b_ref, o_ref, p_ref, *, cn, n):
    c = pl.program_id(0)
    nb = n // cn
    s = c * cn
    lo = jnp.where(c == 0, n - 8, s - 8)
    hi = jnp.where(c == nb - 1, 0, s + cn)
    win = jnp.concatenate(
        [x_ref[pl.ds(lo, 8), :], x_ref[pl.ds(s, cn), :], x_ref[pl.ds(hi, 8), :]],
        axis=0)
    x1w = _lap_valid(win)
    x0c = win[8:cn + 8]
    x2c = 2.0 * _lap_valid(x1w) - x0c
    acc = _dot(x0c, w_ref[0]) + _dot(x1w[4:cn + 4], w_ref[1]) + _dot(x2c, w_ref[2])
    acc = jnp.maximum(acc + b_ref[0, :][None, :], 0.0)
    o_ref[...] = acc
    f = acc.shape[1]
    p = acc.reshape(cn // 4, 4, f)
    p_ref[...] = (p[:, 0] + p[:, 1] + p[:, 2] + p[:, 3]) * 0.25


def _mid_body(p0_ref, w1_ref, b1_ref, w2_ref, b2_ref, w3_ref, b3_ref,
              w4_ref, b4_ref, d0u_ref, d0s_ref, db0_ref, d1u_ref, d1s_ref,
              db1_ref, d2u_ref, d2s_ref, db2_ref, d2_ref):

    def conv_before(x, w_ref, b_ref):
        x1 = _lap_circ(x)
        x2 = 2.0 * _lap_circ(x1) - x
        acc = _dot(x, w_ref[0]) + _dot(x1, w_ref[1]) + _dot(x2, w_ref[2])
        return jnp.maximum(acc + b_ref[0, :][None, :], 0.0)

    def pool(x):
        m, f = x.shape
        p = x.reshape(m // 4, 4, f)
        return (p[:, 0] + p[:, 1] + p[:, 2] + p[:, 3]) * 0.25

    def conv_after_unpool(hc, skip, wu_ref, ws_ref, b_ref):
        ys = []
        for k in range(3):
            u = jnp.repeat(_dot(hc, wu_ref[k]), 4, axis=0)
            ys.append(u + _dot(skip, ws_ref[k]))
        t = _lap_circ(ys[2])
        h = ys[0] - ys[2] + _lap_circ(ys[1] + 2.0 * t)
        return jnp.maximum(h + b_ref[0, :][None, :], 0.0)

    h = conv_before(p0_ref[...], w1_ref, b1_ref)
    skip1 = h
    h = conv_before(pool(h), w2_ref, b2_ref)
    skip2 = h
    h = pool(h)
    hb = [h[:, :128], h[:, 128:]]
    skip3 = []
    d1 = []
    for b in range(2):
        hh = conv_before(hb[b], w3_ref, b3_ref)
        skip3.append(hh)
        hh = conv_before(pool(hh), w4_ref, b4_ref)
        hh = conv_after_unpool(hh, skip3[b], d0u_ref, d0s_ref, db0_ref)
        sk2 = skip2[:, :128] if b == 0 else skip2[:, 128:]
        hh = conv_after_unpool(hh, sk2, d1u_ref, d1s_ref, db1_ref)
        d1.append(hh)
    ys = []
    for k in range(3):
        z = jnp.concatenate([_dot(d1[0], d2u_ref[k]),
                             _dot(d1[1], d2u_ref[k])], axis=1)
        u = jnp.repeat(z, 4, axis=0)
        ys.append(u + _dot(skip1, d2s_ref[k]))
    t = _lap_circ(ys[2])
    h = ys[0] - ys[2] + _lap_circ(ys[1] + 2.0 * t)
    d2_ref[...] = jnp.maximum(h + db2_ref[0, :][None, :], 0.0)


def _fine_body(d2_ref, s0_ref, wu_ref, ws_ref, b3_ref, wo_ref, bo_ref,
               out_ref, *, cn, n):
    c = pl.program_id(0)
    nb = n // cn
    s = c * cn
    nc = n // 4
    cnc = cn // 4
    sc = c * cnc
    lo = jnp.where(c == 0, n - 16, s - 16)
    hi = jnp.where(c == nb - 1, 0, s + cn)
    sw = jnp.concatenate(
        [s0_ref[pl.ds(lo, 16), :], s0_ref[pl.ds(s, cn), :],
         s0_ref[pl.ds(hi, 16), :]], axis=0)
    loc = jnp.where(c == 0, nc - 4, sc - 4)
    hic = jnp.where(c == nb - 1, 0, sc + cnc)
    dw = jnp.concatenate(
        [d2_ref[pl.ds(loc, 4), :], d2_ref[pl.ds(sc, cnc), :],
         d2_ref[pl.ds(hic, 4), :]], axis=0)
    ys = []
    for k in range(3):
        u = jnp.repeat(_dot(dw, wu_ref[k]), 4, axis=0)
        ys.append(u + _dot(sw, ws_ref[k]))
    t = _lap_valid(ys[2])
    u1 = ys[1][4:cn + 28] + 2.0 * t
    h = ys[0][8:cn + 24] - ys[2][8:cn + 24] + _lap_valid(u1)
    h = jnp.maximum(h + b3_ref[0, :][None, :], 0.0)
    z0 = _dot(h[8:cn + 8], wo_ref[0])
    z1 = _dot(h[4:cn + 12], wo_ref[1])
    z2 = _dot(h, wo_ref[2])
    t2 = _lap_valid(z2)
    res = z0 - z2[8:cn + 8] + _lap_valid(z1 + 2.0 * t2)
    out_ref[...] = res + bo_ref[0, :][None, :]


def _pack(x):
    b, n, f = x.shape
    return jnp.transpose(x, (1, 0, 2)).reshape(n, b * f)


def _unpack(xp, f):
    n = xp.shape[0]
    return jnp.transpose(xp.reshape(n, 2, f), (1, 0, 2))


def _pack_w(w):
    eye = jnp.eye(2, dtype=w.dtype)
    return jnp.stack([jnp.kron(eye, w[k]) for k in range(w.shape[0])])


def _pack_b(b):
    return jnp.concatenate([b, b]).reshape(1, -1)


def kernel(x, enc_w0, enc_b0, enc_w1, enc_b1, enc_w2, enc_b2, enc_w3, enc_b3,
           enc_w4, enc_b4, dec_w0, dec_b0, dec_w1, dec_b1, dec_w2, dec_b2,
           dec_w3, dec_b3, out_w, out_b):
    n = 12288
    cn = 2048
    xp = _pack(x)

    skip0, p0 = pl.pallas_call(
        functools.partial(_enc0_body, cn=cn, n=n),
        grid=(n // cn,),
        in_specs=[
            pl.BlockSpec((n, 4), lambda c: (0, 0)),
            pl.BlockSpec((3, 4, 64), lambda c: (0, 0, 0)),
            pl.BlockSpec((1, 64), lambda c: (0, 0)),
        ],
        out_shape=[jax.ShapeDtypeStruct((n, 64), jnp.float32),
                   jax.ShapeDtypeStruct((n // 4, 64), jnp.float32)],
        out_specs=[pl.BlockSpec((cn, 64), lambda c: (c, 0)),
                   pl.BlockSpec((cn // 4, 64), lambda c: (c, 0))],
    )(xp, _pack_w(enc_w0), _pack_b(enc_b0))

    mid_args = (
        p0,
        _pack_w(enc_w1), _pack_b(enc_b1),
        _pack_w(enc_w2), _pack_b(enc_b2),
        enc_w3, enc_b3.reshape(1, -1),
        enc_w4, enc_b4.reshape(1, -1),
        dec_w0[:, :512], dec_w0[:, 512:], dec_b0.reshape(1, -1),
        dec_w1[:, :256], dec_w1[:, 256:], dec_b1.reshape(1, -1),
        dec_w2[:, :128], _pack_w(dec_w2[:, 128:]), _pack_b(dec_b2),
    )
    d2 = pl.pallas_call(
        _mid_body,
        out_shape=jax.ShapeDtypeStruct((3072, 128), jnp.float32),
    )(*mid_args)

    outp = pl.pallas_call(
        functools.partial(_fine_body, cn=cn, n=n),
        grid=(n // cn,),
        in_specs=[
            pl.BlockSpec((n // 4, 128), lambda c: (0, 0)),
            pl.BlockSpec((n, 64), lambda c: (0, 0)),
            pl.BlockSpec((3, 128, 64), lambda c: (0, 0, 0)),
            pl.BlockSpec((3, 64, 64), lambda c: (0, 0, 0)),
            pl.BlockSpec((1, 64), lambda c: (0, 0)),
            pl.BlockSpec((3, 64, 4), lambda c: (0, 0, 0)),
            pl.BlockSpec((1, 4), lambda c: (0, 0)),
        ],
        out_shape=jax.ShapeDtypeStruct((n, 4), jnp.float32),
        out_specs=pl.BlockSpec((cn, 4), lambda c: (c, 0)),
    )(d2, skip0, _pack_w(dec_w3[:, :64]), _pack_w(dec_w3[:, 64:]),
      _pack_b(dec_b3), _pack_w(out_w), _pack_b(out_b))

    return _unpack(outp, 2)
